# baseline (speedup 1.0000x reference)
import functools

import jax
import jax.numpy as jnp
from jax import lax
from jax.experimental import pallas as pl
from jax.experimental.pallas import tpu as pltpu
from jax.experimental.pallas import tpu_sc as plsc

_LANES = 16
_CHUNK = 128


def kernel(inputs, token_table, pos_table):
    B, S = inputs.shape
    V, D = token_table.shape
    N = B * S
    NW = 32
    per_w = N // NW
    K = per_w // _CHUNK
    WPB = S // per_w
    assert N % NW == 0 and per_w % _CHUNK == 0 and S % per_w == 0
    assert D % _LANES == 0

    mesh = plsc.VectorSubcoreMesh(core_axis_name="c", subcore_axis_name="s")

    @functools.partial(
        pl.kernel,
        mesh=mesh,
        out_type=jax.ShapeDtypeStruct((B, S, D), jnp.float32),
        scratch_types=[
            pltpu.VMEM((K, _CHUNK), jnp.int32),
            pltpu.VMEM((K, _CHUNK, D), jnp.float32),
            pltpu.VMEM((K, _CHUNK, D), jnp.float32),
            pltpu.SemaphoreType.DMA,
        ],
        compiler_params=pltpu.CompilerParams(use_tc_tiling_on_sc=False),
    )
    def emb(idx_hbm, tok_hbm, pos_hbm, out_hbm, idx_v, rows_v, pos_v, sem):
        wid = lax.axis_index("s") * 2 + lax.axis_index("c")
        b = wid // WPB
        s0 = (wid % WPB) * per_w

        copies = []
        for j in range(K):
            pltpu.sync_copy(
                idx_hbm.at[b, pl.ds(s0 + j * _CHUNK, _CHUNK)], idx_v.at[j]
            )
            copies.append(
                pltpu.async_copy(tok_hbm.at[idx_v.at[j]], rows_v.at[j], sem)
            )
            copies.append(
                pltpu.async_copy(
                    pos_hbm.at[pl.ds(s0 + j * _CHUNK, _CHUNK)],
                    pos_v.at[j],
                    sem,
                )
            )
        for c in copies:
            c.wait()

        def add_row(r, _):
            for j in range(K):
                for v in range(D // _LANES):
                    sl = pl.ds(v * _LANES, _LANES)
                    rows_v[j, r, sl] = rows_v[j, r, sl] + pos_v[j, r, sl]
            return 0

        lax.fori_loop(0, _CHUNK, add_row, 0)

        for j in range(K):
            pltpu.sync_copy(
                rows_v.at[j], out_hbm.at[b, pl.ds(s0 + j * _CHUNK, _CHUNK)]
            )

    return emb(inputs.astype(jnp.int32), token_table, pos_table)


# speedup vs baseline: 2.1771x; 2.1771x over previous
import functools

import jax
import jax.numpy as jnp
from jax import lax
from jax.experimental import pallas as pl
from jax.experimental.pallas import tpu as pltpu
from jax.experimental.pallas import tpu_sc as plsc

_LANES = 16


def kernel(inputs, token_table, pos_table):
    B, S = inputs.shape
    V, D = token_table.shape
    N = B * S
    NW = 32
    RPW = D // NW
    assert D % NW == 0 and N % _LANES == 0 and S % _LANES == 0

    tok_t = token_table.T
    pos_t = pos_table.T

    mesh = plsc.VectorSubcoreMesh(core_axis_name="c", subcore_axis_name="s")

    @functools.partial(
        pl.kernel,
        mesh=mesh,
        out_type=jax.ShapeDtypeStruct((B, D, S), jnp.float32),
        scratch_types=[
            pltpu.VMEM((N,), jnp.int32),
            pltpu.VMEM((V,), jnp.float32),
            pltpu.VMEM((S,), jnp.float32),
            pltpu.VMEM((N,), jnp.float32),
            pltpu.SemaphoreType.DMA,
        ],
        compiler_params=pltpu.CompilerParams(
            use_tc_tiling_on_sc=True, needs_layout_passes=False
        ),
    )
    def emb(idx_hbm, tok_hbm, pos_hbm, out_hbm, idx_v, row_v, pos_v, out_v, sem):
        wid = lax.axis_index("s") * 2 + lax.axis_index("c")

        for b in range(B):
            pltpu.sync_copy(idx_hbm.at[b], idx_v.at[pl.ds(b * S, S)])

        for r in range(RPW):
            d = wid * RPW + r
            pltpu.sync_copy(tok_hbm.at[d], row_v)
            pltpu.sync_copy(pos_hbm.at[d], pos_v)

            def gather_group(g, _):
                idx16 = idx_v[pl.ds(g * _LANES, _LANES)]
                vals = plsc.load_gather(row_v, [idx16])
                sg = lax.rem(g, S // _LANES)
                pos16 = pos_v[pl.ds(sg * _LANES, _LANES)]
                out_v[pl.ds(g * _LANES, _LANES)] = vals + pos16
                return 0

            lax.fori_loop(0, N // _LANES, gather_group, 0)

            for b in range(B):
                pltpu.sync_copy(
                    out_v.at[pl.ds(b * S, S)], out_hbm.at[b, d]
                )

    out = emb(inputs.astype(jnp.int32), tok_t, pos_t)
    return jnp.swapaxes(out, 1, 2)


# speedup vs baseline: 2.2845x; 1.0493x over previous
import functools

import jax
import jax.numpy as jnp
from jax import lax
from jax.experimental import pallas as pl
from jax.experimental.pallas import tpu as pltpu
from jax.experimental.pallas import tpu_sc as plsc

_LANES = 16


def kernel(inputs, token_table, pos_table):
    B, S = inputs.shape
    V, D = token_table.shape
    N = B * S
    NW = 32
    RPW = D // NW
    assert D % NW == 0 and N % _LANES == 0 and S % _LANES == 0

    tok_t = token_table.T
    pos_t = pos_table.T

    mesh = plsc.VectorSubcoreMesh(core_axis_name="c", subcore_axis_name="s")
    UNROLL = 8

    @functools.partial(
        pl.kernel,
        mesh=mesh,
        out_type=jax.ShapeDtypeStruct((B, D, S), jnp.float32),
        scratch_types=[
            pltpu.VMEM((N,), jnp.int32),
            pltpu.VMEM((V,), jnp.float32),
            pltpu.VMEM((S,), jnp.float32),
            pltpu.VMEM((N,), jnp.float32),
            pltpu.SemaphoreType.DMA,
        ],
        compiler_params=pltpu.CompilerParams(
            use_tc_tiling_on_sc=True, needs_layout_passes=False
        ),
    )
    def emb(idx_hbm, tok_hbm, pos_hbm, out_hbm, idx_v, row_v, pos_v, out_v, sem):
        wid = lax.axis_index("s") * 2 + lax.axis_index("c")

        d0 = wid * RPW
        row_cp = pltpu.async_copy(tok_hbm.at[d0], row_v, sem)
        for b in range(B):
            pltpu.sync_copy(idx_hbm.at[b], idx_v.at[pl.ds(b * S, S)])

        for r in range(RPW):
            d = d0 + r
            row_cp.wait()
            pltpu.sync_copy(pos_hbm.at[d], pos_v)

            def gather_block(blk, _):
                for u in range(UNROLL):
                    g = blk * UNROLL + u
                    idx16 = idx_v[pl.ds(g * _LANES, _LANES)]
                    vals = plsc.load_gather(row_v, [idx16])
                    sg = lax.rem(g, S // _LANES)
                    pos16 = pos_v[pl.ds(sg * _LANES, _LANES)]
                    out_v[pl.ds(g * _LANES, _LANES)] = vals + pos16
                return 0

            lax.fori_loop(0, N // (_LANES * UNROLL), gather_block, 0)

            for b in range(B):
                pltpu.sync_copy(
                    out_v.at[pl.ds(b * S, S)], out_hbm.at[b, d]
                )
            if r + 1 < RPW:
                row_cp = pltpu.async_copy(tok_hbm.at[d + 1], row_v, sem)

    out = emb(inputs.astype(jnp.int32), tok_t, pos_t)
    return jnp.swapaxes(out, 1, 2)
